# Initial kernel scaffold; baseline (speedup 1.0000x reference)
import jax
import jax.numpy as jnp
from jax.experimental import pallas as pl


def kernel(x, weights):
    raise NotImplementedError("write your pallas kernel here")


# speedup vs baseline: 1.1873x; 1.1873x over previous
import functools

import jax
import jax.numpy as jnp
from jax import lax
from jax.experimental import pallas as pl
from jax.experimental.pallas import tpu as pltpu
from jax.experimental.pallas import tpu_sc as plsc

DIM = 32
CHUNK = 128


@functools.cache
def _make(n_chunks: int, dim: int):
    info = plsc.get_sparse_core_info()
    nw = info.num_cores * info.num_subcores
    per_w = n_chunks // nw
    mesh = plsc.VectorSubcoreMesh(core_axis_name="c", subcore_axis_name="s")

    @functools.partial(
        pl.kernel,
        mesh=mesh,
        out_type=jax.ShapeDtypeStruct((n_chunks, CHUNK, dim), jnp.float32),
        scratch_types=[
            pltpu.VMEM((per_w, CHUNK), jnp.int32),
            pltpu.VMEM((CHUNK, dim), jnp.float32),
            pltpu.SemaphoreType.DMA,
        ],
        compiler_params=pltpu.CompilerParams(use_tc_tiling_on_sc=False),
    )
    def emb(idx_hbm, table_hbm, out_hbm, idx_v, rows_v, sem):
        wid = lax.axis_index("s") * info.num_cores + lax.axis_index("c")
        base = wid * per_w
        pltpu.sync_copy(idx_hbm.at[pl.ds(base, per_w)], idx_v)

        def body(j, carry):
            pltpu.async_copy(table_hbm.at[idx_v.at[j]], rows_v, sem).wait()
            pltpu.sync_copy(rows_v, out_hbm.at[base + j])
            return carry

        lax.fori_loop(0, per_w, body, 0)

    return emb


def kernel(x, weights):
    b, s = x.shape
    n = b * s
    idx2d = x.reshape(n // CHUNK, CHUNK).astype(jnp.int32)
    out = _make(n // CHUNK, weights.shape[1])(idx2d, weights)
    return out.reshape(b, s, weights.shape[1])


# speedup vs baseline: 1.3069x; 1.1007x over previous
import functools

import jax
import jax.numpy as jnp
from jax import lax
from jax.experimental import pallas as pl
from jax.experimental.pallas import tpu as pltpu
from jax.experimental.pallas import tpu_sc as plsc

DIM = 32
CHUNK = 128
K = 10


@functools.cache
def _make(n_chunks: int, dim: int):
    info = plsc.get_sparse_core_info()
    nw = info.num_cores * info.num_subcores
    per_w = n_chunks // nw
    ngrp = per_w // K
    npair = ngrp // 2
    mesh = plsc.VectorSubcoreMesh(core_axis_name="c", subcore_axis_name="s")

    @functools.partial(
        pl.kernel,
        mesh=mesh,
        out_type=jax.ShapeDtypeStruct((n_chunks, CHUNK, dim), jnp.float32),
        scratch_types=[
            pltpu.VMEM((per_w, CHUNK), jnp.int32),
            pltpu.VMEM((2, K, CHUNK, dim), jnp.float32),
            pltpu.SemaphoreType.DMA,
            pltpu.SemaphoreType.DMA,
            pltpu.SemaphoreType.DMA,
            pltpu.SemaphoreType.DMA,
        ],
        compiler_params=pltpu.CompilerParams(use_tc_tiling_on_sc=False),
    )
    def emb(idx_hbm, table_hbm, out_hbm, idx_v, rows_v, gsem0, gsem1, wsem0, wsem1):
        wid = lax.axis_index("s") * info.num_cores + lax.axis_index("c")
        base = wid * per_w
        pltpu.sync_copy(idx_hbm.at[pl.ds(base, per_w)], idx_v)
        gsems = (gsem0, gsem1)
        wsems = (wsem0, wsem1)

        def gather_desc(g, p, k):
            return pltpu.make_async_copy(
                table_hbm.at[idx_v.at[g * K + k]], rows_v.at[p, k], gsems[p]
            )

        def wb_desc(g, p, k):
            return pltpu.make_async_copy(
                rows_v.at[p, k], out_hbm.at[base + g * K + k], wsems[p]
            )

        def fire_gathers(g, p):
            for k in range(K):
                gather_desc(g, p, k).start()

        def wait_gathers(g, p):
            for k in range(K):
                gather_desc(g, p, k).wait()

        def fire_wbs(g, p):
            for k in range(K):
                wb_desc(g, p, k).start()

        def wait_wbs(g, p):
            for k in range(K):
                wb_desc(g, p, k).wait()

        fire_gathers(0, 0)

        def body(gg, carry):
            g0 = 2 * gg
            wait_gathers(g0, 0)
            fire_wbs(g0, 0)

            @pl.when(gg > 0)
            def _():
                wait_wbs(g0 - 1, 1)

            fire_gathers(g0 + 1, 1)
            wait_gathers(g0 + 1, 1)
            fire_wbs(g0 + 1, 1)
            wait_wbs(g0, 0)

            @pl.when(gg + 1 < npair)
            def _():
                fire_gathers(g0 + 2, 0)

            return carry

        lax.fori_loop(0, npair, body, 0)
        wait_wbs(ngrp - 1, 1)

    return emb


def kernel(x, weights):
    b, s = x.shape
    n = b * s
    idx2d = x.reshape(n // CHUNK, CHUNK).astype(jnp.int32)
    out = _make(n // CHUNK, weights.shape[1])(idx2d, weights)
    return out.reshape(b, s, weights.shape[1])


# speedup vs baseline: 1.7544x; 1.3424x over previous
import functools

import jax
import jax.numpy as jnp
from jax import lax
from jax.experimental import pallas as pl
from jax.experimental.pallas import tpu as pltpu
from jax.experimental.pallas import tpu_sc as plsc

K = 8


@functools.cache
def _make(n_rows: int, n_cols: int, dim: int):
    info = plsc.get_sparse_core_info()
    nw = info.num_cores * info.num_subcores
    per_w = n_rows // nw
    ngrp = per_w // K
    npair = ngrp // 2
    mesh = plsc.VectorSubcoreMesh(core_axis_name="c", subcore_axis_name="s")

    @functools.partial(
        pl.kernel,
        mesh=mesh,
        out_type=jax.ShapeDtypeStruct((n_rows, n_cols, dim), jnp.float32),
        scratch_types=[
            pltpu.VMEM((per_w, n_cols), jnp.int32),
            pltpu.VMEM((2, K, n_cols, dim), jnp.float32),
            pltpu.SemaphoreType.DMA,
            pltpu.SemaphoreType.DMA,
            pltpu.SemaphoreType.DMA,
            pltpu.SemaphoreType.DMA,
        ],
        compiler_params=pltpu.CompilerParams(use_tc_tiling_on_sc=False),
    )
    def emb(idx_hbm, table_hbm, out_hbm, idx_v, rows_v, gsem0, gsem1, wsem0, wsem1):
        wid = lax.axis_index("s") * info.num_cores + lax.axis_index("c")
        base = wid * per_w
        pltpu.sync_copy(idx_hbm.at[pl.ds(base, per_w)], idx_v)
        gsems = (gsem0, gsem1)
        wsems = (wsem0, wsem1)

        def gather_desc(g, p, k):
            return pltpu.make_async_copy(
                table_hbm.at[idx_v.at[g * K + k]], rows_v.at[p, k], gsems[p]
            )

        def wb_desc(g, p, k):
            return pltpu.make_async_copy(
                rows_v.at[p, k], out_hbm.at[base + g * K + k], wsems[p]
            )

        def fire_gathers(g, p):
            for k in range(K):
                gather_desc(g, p, k).start()

        def wait_gathers(g, p):
            for k in range(K):
                gather_desc(g, p, k).wait()

        def fire_wbs(g, p):
            for k in range(K):
                wb_desc(g, p, k).start()

        def wait_wbs(g, p):
            for k in range(K):
                wb_desc(g, p, k).wait()

        fire_gathers(0, 0)

        def body(gg, carry):
            g0 = 2 * gg
            wait_gathers(g0, 0)
            fire_wbs(g0, 0)

            @pl.when(gg > 0)
            def _():
                wait_wbs(g0 - 1, 1)

            fire_gathers(g0 + 1, 1)
            wait_gathers(g0 + 1, 1)
            fire_wbs(g0 + 1, 1)
            wait_wbs(g0, 0)

            @pl.when(gg + 1 < npair)
            def _():
                fire_gathers(g0 + 2, 0)

            return carry

        lax.fori_loop(0, npair, body, 0)
        wait_wbs(ngrp - 1, 1)

    return emb


def kernel(x, weights):
    b, s = x.shape
    out = _make(b, s, weights.shape[1])(x.astype(jnp.int32), weights)
    return out
